# baseline (speedup 1.0000x reference)
import functools

import jax
import jax.numpy as jnp
from jax import lax
from jax.experimental import pallas as pl
from jax.experimental.pallas import tpu as pltpu
from jax.experimental.pallas import tpu_sc as plsc

N = 10000
D = 128
DH = D // 2
E_MP = 320000
E_SUP = 100000

NC = 2
NS = 16
NW = NC * NS

CH = 128
NCH = 80
EPW = NCH * CH
E_MP_PAD = NW * EPW

TRASH = 10240
N_ACC = 10496
APS = N_ACC // NS
N_OUT = 10240
RPS = N_OUT // NS
ZR = 128
KA = 4
KC = 2

SCH2 = 64
NSCH = 56
SPW = NSCH * SCH2
E_SUP_PAD = NW * SPW


def _mesh():
    return plsc.VectorSubcoreMesh(core_axis_name="c", subcore_axis_name="s")


_SC_PARAMS = pltpu.CompilerParams(use_tc_tiling_on_sc=False)


def _fill_2d(buf, rows, cols, val):
    v = jnp.full((16,), val, jnp.float32)

    def body(i, carry):
        r = i // (cols // 16)
        col = (i % (cols // 16)) * 16
        buf[r, pl.ds(col, 16)] = v
        return carry

    lax.fori_loop(0, rows * (cols // 16), body, 0)


def _fill_1d(buf, n, val):
    v = jnp.full((16,), val, jnp.float32)

    def body(i, carry):
        buf[pl.ds(i * 16, 16)] = v
        return carry

    lax.fori_loop(0, n // 16, body, 0)


def _zero_acc(zsrc, acc_sh, s):
    base = s * APS
    for k in range(APS // ZR):
        pltpu.sync_copy(zsrc, acc_sh.at[pl.ds(base + k * ZR, ZR)])
    rem = APS % ZR
    if rem:
        pltpu.sync_copy(zsrc.at[pl.ds(0, rem)],
                        acc_sh.at[pl.ds(base + (APS // ZR) * ZR, rem)])


def _agg_pipeline(x_hbm, idx_s, idx_d, bufs_a, bufs_b, acc_sh,
                  gsem, ssem_a, ssem_b, k):
    npairs = NCH // (2 * k)

    def run_group(t, g_off, bufs, ssem):
        g = 2 * t + g_off

        @pl.when(t > 0)
        def _():
            for b in range(k):
                pltpu.make_async_copy(x_hbm.at[pl.ds(0, CH)], bufs[b],
                                      ssem).wait()
        descs = []
        for b in range(k):
            j = g * k + b
            descs.append(
                pltpu.async_copy(x_hbm.at[idx_s.at[j]], bufs[b], gsem))
        for d in descs:
            d.wait()
        for b in range(k):
            j = g * k + b
            pltpu.async_copy(bufs[b], acc_sh.at[idx_d.at[j]], ssem, add=True)

    def pair(t, carry):
        run_group(t, 0, bufs_a, ssem_a)
        run_group(t, 1, bufs_b, ssem_b)
        return carry

    lax.fori_loop(0, npairs, pair, 0)
    for b in range(k):
        pltpu.make_async_copy(x_hbm.at[pl.ds(0, CH)], bufs_a[b], ssem_a).wait()
        pltpu.make_async_copy(x_hbm.at[pl.ds(0, CH)], bufs_b[b], ssem_b).wait()


def _agg_count_body(x_hbm, src_hbm, dst_hbm, part_hbm, cnt_hbm,
                    idx_s, idx_d, zed, ones_v, a0, a1, a2, a3, b0, b1, b2, b3,
                    acc_sh, cnt_sh, gsem, ssem_a, ssem_b, csem):
    c = lax.axis_index("c")
    s = lax.axis_index("s")
    w = c * NS + s

    _fill_1d(zed, APS, 0.0)
    _fill_1d(ones_v, 128, 1.0)
    pltpu.sync_copy(zed, cnt_sh.at[pl.ds(s * APS, APS)])
    _fill_2d(a0, ZR, DH, 0.0)
    _zero_acc(a0, acc_sh, s)

    pltpu.sync_copy(src_hbm.at[w], idx_s)
    pltpu.sync_copy(dst_hbm.at[w], idx_d)

    plsc.subcore_barrier()

    def cfire(j, carry):
        pltpu.async_copy(ones_v.at[pl.ds(0, CH)], cnt_sh.at[idx_d.at[j]],
                         csem, add=True)
        return carry

    lax.fori_loop(0, NCH, cfire, 0)

    _agg_pipeline(x_hbm, idx_s, idx_d, (a0, a1, a2, a3), (b0, b1, b2, b3),
                  acc_sh, gsem, ssem_a, ssem_b, KA)

    def cdrain(j, carry):
        pltpu.make_async_copy(x_hbm.at[pl.ds(0, 2)], a0.at[pl.ds(0, 2)],
                              csem).wait()
        return carry

    lax.fori_loop(0, NCH, cdrain, 0)

    plsc.subcore_barrier()

    pltpu.sync_copy(acc_sh.at[pl.ds(s * RPS, RPS)],
                    part_hbm.at[c, pl.ds(s * RPS, RPS)])
    pltpu.sync_copy(cnt_sh.at[pl.ds(s * RPS, RPS)],
                    cnt_hbm.at[pl.ds(c * N_OUT + s * RPS, RPS)])


def _agg_body(x_hbm, src_hbm, dst_hbm, part_hbm,
              idx_s, idx_d, a0, a1, a2, a3, b0, b1, b2, b3, acc_sh,
              gsem, ssem_a, ssem_b):
    c = lax.axis_index("c")
    s = lax.axis_index("s")
    w = c * NS + s

    _fill_2d(a0, ZR, DH, 0.0)
    _zero_acc(a0, acc_sh, s)

    pltpu.sync_copy(src_hbm.at[w], idx_s)
    pltpu.sync_copy(dst_hbm.at[w], idx_d)

    plsc.subcore_barrier()

    _agg_pipeline(x_hbm, idx_s, idx_d, (a0, a1, a2, a3), (b0, b1, b2, b3),
                  acc_sh, gsem, ssem_a, ssem_b, KA)

    plsc.subcore_barrier()

    pltpu.sync_copy(acc_sh.at[pl.ds(s * RPS, RPS)],
                    part_hbm.at[c, pl.ds(s * RPS, RPS)])


def _row_buf():
    return pltpu.VMEM((CH, DH), jnp.float32)


_agg_count = functools.partial(
    pl.kernel,
    out_type=[jax.ShapeDtypeStruct((NC, N_OUT, DH), jnp.float32),
              jax.ShapeDtypeStruct((NC * N_OUT,), jnp.float32)],
    mesh=_mesh(),
    compiler_params=_SC_PARAMS,
    scratch_types=[
        pltpu.VMEM((NCH, CH), jnp.int32),
        pltpu.VMEM((NCH, CH), jnp.int32),
        pltpu.VMEM((APS,), jnp.float32),
        pltpu.VMEM((128,), jnp.float32),
        _row_buf(), _row_buf(), _row_buf(), _row_buf(),
        _row_buf(), _row_buf(), _row_buf(), _row_buf(),
        pltpu.VMEM_SHARED((N_ACC, DH), jnp.float32),
        pltpu.VMEM_SHARED((N_ACC,), jnp.float32),
        pltpu.SemaphoreType.DMA,
        pltpu.SemaphoreType.DMA,
        pltpu.SemaphoreType.DMA,
        pltpu.SemaphoreType.DMA,
    ],
)(_agg_count_body)

_agg = functools.partial(
    pl.kernel,
    out_type=jax.ShapeDtypeStruct((NC, N_OUT, DH), jnp.float32),
    mesh=_mesh(),
    compiler_params=_SC_PARAMS,
    scratch_types=[
        pltpu.VMEM((NCH, CH), jnp.int32),
        pltpu.VMEM((NCH, CH), jnp.int32),
        _row_buf(), _row_buf(), _row_buf(), _row_buf(),
        _row_buf(), _row_buf(), _row_buf(), _row_buf(),
        pltpu.VMEM_SHARED((N_ACC, DH), jnp.float32),
        pltpu.SemaphoreType.DMA,
        pltpu.SemaphoreType.DMA,
        pltpu.SemaphoreType.DMA,
    ],
)(_agg_body)


def _sup_gather_body(h_hbm, ssrc_hbm, sdst_hbm, osrc_hbm, odst_hbm,
                     idx_s, idx_d, p0s, p0d, p1s, p1d, gsem, wsem_a, wsem_b):
    c = lax.axis_index("c")
    s = lax.axis_index("s")
    w = c * NS + s
    base = w * SPW

    pltpu.sync_copy(ssrc_hbm.at[w], idx_s)
    pltpu.sync_copy(sdst_hbm.at[w], idx_d)

    def run_chunk(t, g_off, bs, bd, wsem):
        j = 2 * t + g_off

        @pl.when(t > 0)
        def _():
            pltpu.make_async_copy(h_hbm.at[pl.ds(0, SCH2)], bs, wsem).wait()
            pltpu.make_async_copy(h_hbm.at[pl.ds(0, SCH2)], bd, wsem).wait()
        d1 = pltpu.async_copy(h_hbm.at[idx_s.at[j]], bs, gsem)
        d2 = pltpu.async_copy(h_hbm.at[idx_d.at[j]], bd, gsem)
        d1.wait()
        d2.wait()
        pltpu.async_copy(bs, osrc_hbm.at[pl.ds(base + j * SCH2, SCH2)], wsem)
        pltpu.async_copy(bd, odst_hbm.at[pl.ds(base + j * SCH2, SCH2)], wsem)

    def pair(t, carry):
        run_chunk(t, 0, p0s, p0d, wsem_a)
        run_chunk(t, 1, p1s, p1d, wsem_b)
        return carry

    lax.fori_loop(0, NSCH // 2, pair, 0)
    pltpu.make_async_copy(h_hbm.at[pl.ds(0, SCH2)], p0s, wsem_a).wait()
    pltpu.make_async_copy(h_hbm.at[pl.ds(0, SCH2)], p0d, wsem_a).wait()
    pltpu.make_async_copy(h_hbm.at[pl.ds(0, SCH2)], p1s, wsem_b).wait()
    pltpu.make_async_copy(h_hbm.at[pl.ds(0, SCH2)], p1d, wsem_b).wait()


_sup_gather = functools.partial(
    pl.kernel,
    out_type=[jax.ShapeDtypeStruct((E_SUP_PAD, D), jnp.float32),
              jax.ShapeDtypeStruct((E_SUP_PAD, D), jnp.float32)],
    mesh=_mesh(),
    compiler_params=_SC_PARAMS,
    scratch_types=[
        pltpu.VMEM((NSCH, SCH2), jnp.int32),
        pltpu.VMEM((NSCH, SCH2), jnp.int32),
        pltpu.VMEM((SCH2, D), jnp.float32),
        pltpu.VMEM((SCH2, D), jnp.float32),
        pltpu.VMEM((SCH2, D), jnp.float32),
        pltpu.VMEM((SCH2, D), jnp.float32),
        pltpu.SemaphoreType.DMA,
        pltpu.SemaphoreType.DMA,
        pltpu.SemaphoreType.DMA,
    ],
)(_sup_gather_body)


_BR = 1000


def _sage_tc(pA, pB, cnt2, x, Wl, bl, Wr, relu, emit_halves):
    def body(pA0, pA1, pB0, pB1, cnt_ref, x_ref, wl_ref, bl_ref, wr_ref, *outs):
        cnt = cnt_ref[:, 0] + cnt_ref[:, 1]
        recip = 1.0 / jnp.maximum(cnt, 1.0)
        agg = jnp.concatenate([pA0[0] + pA1[0], pB0[0] + pB1[0]], axis=1)
        mean = agg * recip[:, None]
        h = lax.dot_general(mean, wl_ref[...], (((1,), (1,)), ((), ())),
                            preferred_element_type=jnp.float32)
        h = h + bl_ref[...]
        h = h + lax.dot_general(x_ref[...], wr_ref[...], (((1,), (1,)), ((), ())),
                                preferred_element_type=jnp.float32)
        if relu:
            h = jnp.maximum(h, 0.0)
        outs[0][...] = h
        if emit_halves:
            outs[1][...] = h[:, :DH]
            outs[2][...] = h[:, DH:]

    grid = (N // _BR,)
    out_specs = [pl.BlockSpec((_BR, D), lambda i: (i, 0))]
    out_shape = [jax.ShapeDtypeStruct((N, D), jnp.float32)]
    if emit_halves:
        out_specs += [pl.BlockSpec((_BR, DH), lambda i: (i, 0))] * 2
        out_shape += [jax.ShapeDtypeStruct((N, DH), jnp.float32)] * 2
    return pl.pallas_call(
        body,
        grid=grid,
        in_specs=[
            pl.BlockSpec((1, _BR, DH), lambda i: (0, i, 0)),
            pl.BlockSpec((1, _BR, DH), lambda i: (1, i, 0)),
            pl.BlockSpec((1, _BR, DH), lambda i: (0, i, 0)),
            pl.BlockSpec((1, _BR, DH), lambda i: (1, i, 0)),
            pl.BlockSpec((_BR, NC), lambda i: (i, 0)),
            pl.BlockSpec((_BR, D), lambda i: (i, 0)),
            pl.BlockSpec((D, D), lambda i: (0, 0)),
            pl.BlockSpec((1, D), lambda i: (0, 0)),
            pl.BlockSpec((D, D), lambda i: (0, 0)),
        ],
        out_specs=out_specs,
        out_shape=out_shape,
    )(pA, pA, pB, pB, cnt2, x, Wl, bl, Wr)


_BS = 4000


def _dot_tc(a, b):
    def body(a_ref, b_ref, o_ref):
        o_ref[...] = jnp.sum(a_ref[...] * b_ref[...], axis=1, keepdims=True)

    grid = (E_SUP // _BS,)
    return pl.pallas_call(
        body,
        grid=grid,
        in_specs=[
            pl.BlockSpec((_BS, D), lambda i: (i, 0)),
            pl.BlockSpec((_BS, D), lambda i: (i, 0)),
        ],
        out_specs=pl.BlockSpec((_BS, 1), lambda i: (i, 0)),
        out_shape=jax.ShapeDtypeStruct((E_SUP, 1), jnp.float32),
    )(a, b)


def kernel(node_embeddings, message_passing_edge_index, supervision_edge_index,
           Wl1, bl1, Wr1, Wl2, bl2, Wr2, Wl3, bl3, Wr3):
    mp_pad = jnp.concatenate(
        [message_passing_edge_index,
         jnp.stack([jnp.zeros((E_MP_PAD - E_MP,), jnp.int32),
                    jnp.full((E_MP_PAD - E_MP,), TRASH, jnp.int32)])], axis=1)
    src = mp_pad[0].reshape(NW, NCH, CH)
    dst = mp_pad[1].reshape(NW, NCH, CH)
    sup_pad = jnp.concatenate(
        [supervision_edge_index,
         jnp.zeros((2, E_SUP_PAD - E_SUP), jnp.int32)], axis=1)
    ssrc = sup_pad[0].reshape(NW, NSCH, SCH2)
    sdst = sup_pad[1].reshape(NW, NSCH, SCH2)

    x = node_embeddings
    xA = x[:, :DH]
    xB = x[:, DH:]

    pA, cnt = _agg_count(xA, src, dst)
    pB = _agg(xB, src, dst)
    cnt2 = cnt.reshape(NC, N_OUT)[:, :N].T

    h, hA, hB = _sage_tc(pA, pB, cnt2, x, Wl1, bl1.reshape(1, D), Wr1,
                         True, True)
    pA = _agg(hA, src, dst)
    pB = _agg(hB, src, dst)
    h, hA, hB = _sage_tc(pA, pB, cnt2, h, Wl2, bl2.reshape(1, D), Wr2,
                         True, True)
    pA = _agg(hA, src, dst)
    pB = _agg(hB, src, dst)
    h = _sage_tc(pA, pB, cnt2, h, Wl3, bl3.reshape(1, D), Wr3, False, False)[0]

    src_rows, dst_rows = _sup_gather(h, ssrc, sdst)
    scores = _dot_tc(src_rows, dst_rows)
    return scores.reshape(E_SUP)


# speedup vs baseline: 4.3166x; 4.3166x over previous
import functools

import jax
import jax.numpy as jnp
from jax import lax
from jax.experimental import pallas as pl
from jax.experimental.pallas import tpu as pltpu
from jax.experimental.pallas import tpu_sc as plsc

N = 10000
D = 128
DH = D // 2
E_MP = 320000
E_SUP = 100000

NC = 2
NS = 16
NW = NC * NS

CH = 100
NCH = 100
EPW = NCH * CH

N_ACC = 10240
APS = N_ACC // NS
N_OUT = 10240
RPS = N_OUT // NS
ZR = 128
KA = 5

SCH2 = 56
NSCH = 56
SPW = NSCH * SCH2
E_SUP_PAD = NW * SPW


def _mesh():
    return plsc.VectorSubcoreMesh(core_axis_name="c", subcore_axis_name="s")


_SC_PARAMS = pltpu.CompilerParams(use_tc_tiling_on_sc=False)


def _fill_2d(buf, rows, cols, val):
    v = jnp.full((16,), val, jnp.float32)

    def body(i, carry):
        r = i // (cols // 16)
        col = (i % (cols // 16)) * 16
        buf[r, pl.ds(col, 16)] = v
        return carry

    lax.fori_loop(0, rows * (cols // 16), body, 0)


def _fill_1d(buf, n, val):
    v = jnp.full((16,), val, jnp.float32)

    def body(i, carry):
        buf[pl.ds(i * 16, 16)] = v
        return carry

    lax.fori_loop(0, n // 16, body, 0)


def _zero_acc(zsrc, acc_sh, s):
    base = s * APS
    for k in range(APS // CH):
        pltpu.sync_copy(zsrc, acc_sh.at[pl.ds(base + k * CH, CH)])
    rem = APS % CH
    if rem:
        pltpu.sync_copy(zsrc.at[pl.ds(0, rem)],
                        acc_sh.at[pl.ds(base + (APS // CH) * CH, rem)])


def _agg_pipeline(x_hbm, idx_s, idx_d, bufs_a, bufs_b, acc_sh,
                  gsem, ssem_a, ssem_b, k):
    npairs = NCH // (2 * k)

    def run_group(t, g_off, bufs, ssem):
        g = 2 * t + g_off

        @pl.when(t > 0)
        def _():
            for b in range(k):
                pltpu.make_async_copy(x_hbm.at[pl.ds(0, CH)], bufs[b],
                                      ssem).wait()
        descs = []
        for b in range(k):
            j = g * k + b
            descs.append(
                pltpu.async_copy(x_hbm.at[idx_s.at[j]], bufs[b], gsem))
        for d in descs:
            d.wait()
        for b in range(k):
            j = g * k + b
            pltpu.async_copy(bufs[b], acc_sh.at[idx_d.at[j]], ssem, add=True)

    def pair(t, carry):
        run_group(t, 0, bufs_a, ssem_a)
        run_group(t, 1, bufs_b, ssem_b)
        return carry

    lax.fori_loop(0, npairs, pair, 0)
    for b in range(k):
        pltpu.make_async_copy(x_hbm.at[pl.ds(0, CH)], bufs_a[b], ssem_a).wait()
        pltpu.make_async_copy(x_hbm.at[pl.ds(0, CH)], bufs_b[b], ssem_b).wait()


def _agg_count_body(x_hbm, src_hbm, dst_hbm, part_hbm, cnt_hbm,
                    idx_s, idx_d, zed, ones_v,
                    a0, a1, a2, a3, a4, b0, b1, b2, b3, b4,
                    acc_sh, cnt_sh, gsem, ssem_a, ssem_b, csem):
    c = lax.axis_index("c")
    s = lax.axis_index("s")
    w = c * NS + s

    _fill_1d(zed, APS, 0.0)
    _fill_1d(ones_v, 128, 1.0)
    pltpu.sync_copy(zed, cnt_sh.at[pl.ds(s * APS, APS)])
    _fill_2d(a0, CH, DH, 0.0)
    _zero_acc(a0, acc_sh, s)

    pltpu.sync_copy(src_hbm.at[w], idx_s)
    pltpu.sync_copy(dst_hbm.at[w], idx_d)

    plsc.subcore_barrier()

    def cfire(j, carry):
        pltpu.async_copy(ones_v.at[pl.ds(0, CH)], cnt_sh.at[idx_d.at[j]],
                         csem, add=True)
        return carry

    lax.fori_loop(0, NCH, cfire, 0)

    _agg_pipeline(x_hbm, idx_s, idx_d, (a0, a1, a2, a3, a4),
                  (b0, b1, b2, b3, b4), acc_sh, gsem, ssem_a, ssem_b, KA)

    def cdrain(j, carry):
        pltpu.make_async_copy(cnt_hbm.at[pl.ds(0, CH)], zed.at[pl.ds(0, CH)],
                              csem).wait()
        return carry

    lax.fori_loop(0, NCH, cdrain, 0)

    plsc.subcore_barrier()

    pltpu.sync_copy(acc_sh.at[pl.ds(s * RPS, RPS)],
                    part_hbm.at[c, pl.ds(s * RPS, RPS)])
    pltpu.sync_copy(cnt_sh.at[pl.ds(s * RPS, RPS)],
                    cnt_hbm.at[pl.ds(c * N_OUT + s * RPS, RPS)])


def _agg_body(x_hbm, src_hbm, dst_hbm, part_hbm,
              idx_s, idx_d, a0, a1, a2, a3, a4, b0, b1, b2, b3, b4, acc_sh,
              gsem, ssem_a, ssem_b):
    c = lax.axis_index("c")
    s = lax.axis_index("s")
    w = c * NS + s

    _fill_2d(a0, CH, DH, 0.0)
    _zero_acc(a0, acc_sh, s)

    pltpu.sync_copy(src_hbm.at[w], idx_s)
    pltpu.sync_copy(dst_hbm.at[w], idx_d)

    plsc.subcore_barrier()

    _agg_pipeline(x_hbm, idx_s, idx_d, (a0, a1, a2, a3, a4),
                  (b0, b1, b2, b3, b4), acc_sh, gsem, ssem_a, ssem_b, KA)

    plsc.subcore_barrier()

    pltpu.sync_copy(acc_sh.at[pl.ds(s * RPS, RPS)],
                    part_hbm.at[c, pl.ds(s * RPS, RPS)])


def _row_buf():
    return pltpu.VMEM((CH, DH), jnp.float32)


_agg_count = functools.partial(
    pl.kernel,
    out_type=[jax.ShapeDtypeStruct((NC, N_OUT, DH), jnp.float32),
              jax.ShapeDtypeStruct((NC * N_OUT,), jnp.float32)],
    mesh=_mesh(),
    compiler_params=_SC_PARAMS,
    scratch_types=[
        pltpu.VMEM((NCH, CH), jnp.int32),
        pltpu.VMEM((NCH, CH), jnp.int32),
        pltpu.VMEM((APS,), jnp.float32),
        pltpu.VMEM((128,), jnp.float32),
        _row_buf(), _row_buf(), _row_buf(), _row_buf(), _row_buf(),
        _row_buf(), _row_buf(), _row_buf(), _row_buf(), _row_buf(),
        pltpu.VMEM_SHARED((N_ACC, DH), jnp.float32),
        pltpu.VMEM_SHARED((N_ACC,), jnp.float32),
        pltpu.SemaphoreType.DMA,
        pltpu.SemaphoreType.DMA,
        pltpu.SemaphoreType.DMA,
        pltpu.SemaphoreType.DMA,
    ],
)(_agg_count_body)

_agg = functools.partial(
    pl.kernel,
    out_type=jax.ShapeDtypeStruct((NC, N_OUT, DH), jnp.float32),
    mesh=_mesh(),
    compiler_params=_SC_PARAMS,
    scratch_types=[
        pltpu.VMEM((NCH, CH), jnp.int32),
        pltpu.VMEM((NCH, CH), jnp.int32),
        _row_buf(), _row_buf(), _row_buf(), _row_buf(), _row_buf(),
        _row_buf(), _row_buf(), _row_buf(), _row_buf(), _row_buf(),
        pltpu.VMEM_SHARED((N_ACC, DH), jnp.float32),
        pltpu.SemaphoreType.DMA,
        pltpu.SemaphoreType.DMA,
        pltpu.SemaphoreType.DMA,
    ],
)(_agg_body)


def _sup_gather_body(h_hbm, ssrc_hbm, sdst_hbm, osrc_hbm, odst_hbm,
                     idx_s, idx_d, p0s, p0d, p1s, p1d, gsem, wsem_a, wsem_b):
    c = lax.axis_index("c")
    s = lax.axis_index("s")
    w = c * NS + s
    base = w * SPW

    pltpu.sync_copy(ssrc_hbm.at[w], idx_s)
    pltpu.sync_copy(sdst_hbm.at[w], idx_d)

    def run_chunk(t, g_off, bs, bd, wsem):
        j = 2 * t + g_off

        @pl.when(t > 0)
        def _():
            pltpu.make_async_copy(h_hbm.at[pl.ds(0, SCH2)], bs, wsem).wait()
            pltpu.make_async_copy(h_hbm.at[pl.ds(0, SCH2)], bd, wsem).wait()
        d1 = pltpu.async_copy(h_hbm.at[idx_s.at[j]], bs, gsem)
        d2 = pltpu.async_copy(h_hbm.at[idx_d.at[j]], bd, gsem)
        d1.wait()
        d2.wait()
        pltpu.async_copy(bs, osrc_hbm.at[pl.ds(base + j * SCH2, SCH2)], wsem)
        pltpu.async_copy(bd, odst_hbm.at[pl.ds(base + j * SCH2, SCH2)], wsem)

    def pair(t, carry):
        run_chunk(t, 0, p0s, p0d, wsem_a)
        run_chunk(t, 1, p1s, p1d, wsem_b)
        return carry

    lax.fori_loop(0, NSCH // 2, pair, 0)
    pltpu.make_async_copy(h_hbm.at[pl.ds(0, SCH2)], p0s, wsem_a).wait()
    pltpu.make_async_copy(h_hbm.at[pl.ds(0, SCH2)], p0d, wsem_a).wait()
    pltpu.make_async_copy(h_hbm.at[pl.ds(0, SCH2)], p1s, wsem_b).wait()
    pltpu.make_async_copy(h_hbm.at[pl.ds(0, SCH2)], p1d, wsem_b).wait()


_sup_gather = functools.partial(
    pl.kernel,
    out_type=[jax.ShapeDtypeStruct((E_SUP_PAD, D), jnp.float32),
              jax.ShapeDtypeStruct((E_SUP_PAD, D), jnp.float32)],
    mesh=_mesh(),
    compiler_params=_SC_PARAMS,
    scratch_types=[
        pltpu.VMEM((NSCH, SCH2), jnp.int32),
        pltpu.VMEM((NSCH, SCH2), jnp.int32),
        pltpu.VMEM((SCH2, D), jnp.float32),
        pltpu.VMEM((SCH2, D), jnp.float32),
        pltpu.VMEM((SCH2, D), jnp.float32),
        pltpu.VMEM((SCH2, D), jnp.float32),
        pltpu.SemaphoreType.DMA,
        pltpu.SemaphoreType.DMA,
        pltpu.SemaphoreType.DMA,
    ],
)(_sup_gather_body)


_BR = 1000


def _sage_tc(pA, pB, cnt2, x, Wl, bl, Wr, relu, emit_halves):
    def body(pA0, pA1, pB0, pB1, cnt_ref, x_ref, wl_ref, bl_ref, wr_ref, *outs):
        cnt = cnt_ref[:, 0] + cnt_ref[:, 1]
        recip = 1.0 / jnp.maximum(cnt, 1.0)
        agg = jnp.concatenate([pA0[0] + pA1[0], pB0[0] + pB1[0]], axis=1)
        mean = agg * recip[:, None]
        h = lax.dot_general(mean, wl_ref[...], (((1,), (1,)), ((), ())),
                            preferred_element_type=jnp.float32)
        h = h + bl_ref[...]
        h = h + lax.dot_general(x_ref[...], wr_ref[...], (((1,), (1,)), ((), ())),
                                preferred_element_type=jnp.float32)
        if relu:
            h = jnp.maximum(h, 0.0)
        outs[0][...] = h
        if emit_halves:
            outs[1][...] = h[:, :DH]
            outs[2][...] = h[:, DH:]

    grid = (N // _BR,)
    out_specs = [pl.BlockSpec((_BR, D), lambda i: (i, 0))]
    out_shape = [jax.ShapeDtypeStruct((N, D), jnp.float32)]
    if emit_halves:
        out_specs += [pl.BlockSpec((_BR, DH), lambda i: (i, 0))] * 2
        out_shape += [jax.ShapeDtypeStruct((N, DH), jnp.float32)] * 2
    return pl.pallas_call(
        body,
        grid=grid,
        in_specs=[
            pl.BlockSpec((1, _BR, DH), lambda i: (0, i, 0)),
            pl.BlockSpec((1, _BR, DH), lambda i: (1, i, 0)),
            pl.BlockSpec((1, _BR, DH), lambda i: (0, i, 0)),
            pl.BlockSpec((1, _BR, DH), lambda i: (1, i, 0)),
            pl.BlockSpec((_BR, NC), lambda i: (i, 0)),
            pl.BlockSpec((_BR, D), lambda i: (i, 0)),
            pl.BlockSpec((D, D), lambda i: (0, 0)),
            pl.BlockSpec((1, D), lambda i: (0, 0)),
            pl.BlockSpec((D, D), lambda i: (0, 0)),
        ],
        out_specs=out_specs,
        out_shape=out_shape,
    )(pA, pA, pB, pB, cnt2, x, Wl, bl, Wr)


_BS = 4000


def _dot_tc(a, b):
    def body(a_ref, b_ref, o_ref):
        o_ref[...] = jnp.sum(a_ref[...] * b_ref[...], axis=1, keepdims=True)

    grid = (E_SUP // _BS,)
    return pl.pallas_call(
        body,
        grid=grid,
        in_specs=[
            pl.BlockSpec((_BS, D), lambda i: (i, 0)),
            pl.BlockSpec((_BS, D), lambda i: (i, 0)),
        ],
        out_specs=pl.BlockSpec((_BS, 1), lambda i: (i, 0)),
        out_shape=jax.ShapeDtypeStruct((E_SUP, 1), jnp.float32),
    )(a, b)


def kernel(node_embeddings, message_passing_edge_index, supervision_edge_index,
           Wl1, bl1, Wr1, Wl2, bl2, Wr2, Wl3, bl3, Wr3):
    src = message_passing_edge_index[0].reshape(NW, NCH, CH)
    dst = message_passing_edge_index[1].reshape(NW, NCH, CH)
    pad_idx = jnp.arange(E_SUP_PAD - E_SUP, dtype=jnp.int32) % N
    sup_pad = jnp.concatenate(
        [supervision_edge_index, jnp.stack([pad_idx, pad_idx])], axis=1)
    ssrc = sup_pad[0].reshape(NW, NSCH, SCH2)
    sdst = sup_pad[1].reshape(NW, NSCH, SCH2)

    x = node_embeddings
    xA = x[:, :DH]
    xB = x[:, DH:]

    pA, cnt = _agg_count(xA, src, dst)
    pB = _agg(xB, src, dst)
    cnt2 = cnt.reshape(NC, N_OUT)[:, :N].T

    h, hA, hB = _sage_tc(pA, pB, cnt2, x, Wl1, bl1.reshape(1, D), Wr1,
                         True, True)
    pA = _agg(hA, src, dst)
    pB = _agg(hB, src, dst)
    h, hA, hB = _sage_tc(pA, pB, cnt2, h, Wl2, bl2.reshape(1, D), Wr2,
                         True, True)
    pA = _agg(hA, src, dst)
    pB = _agg(hB, src, dst)
    h = _sage_tc(pA, pB, cnt2, h, Wl3, bl3.reshape(1, D), Wr3, False, False)[0]

    src_rows, dst_rows = _sup_gather(h, ssrc, sdst)
    scores = _dot_tc(src_rows, dst_rows)
    return scores.reshape(E_SUP)
